# baseline (device time: 71285 ns/iter reference)
import jax
import jax.numpy as jnp
from jax import lax
from jax.experimental import pallas as pl
from jax.experimental.pallas import tpu as pltpu

W = 32
H = 1


def kernel(A, B):
    M, K = A.shape
    _, N = B.shape
    rows = M // W
    nh = N // H

    def body(
        a_ref, b_ref, out_ref, sendbuf, recvbuf, accbuf,
        send1, recv1, send2, recv2,
    ):
        me = lax.axis_index("i")

        part = jnp.dot(
            a_ref[...].astype(jnp.bfloat16),
            b_ref[...].astype(jnp.bfloat16),
            preferred_element_type=jnp.float32,
        )
        sendbuf[...] = part.astype(jnp.bfloat16)

        def p1_send(h):
            def step_fn(step, _):
                tgt = lax.rem(me + step, W)
                pltpu.make_async_remote_copy(
                    src_ref=sendbuf.at[pl.ds(tgt * rows, rows), pl.ds(h * nh, nh)],
                    dst_ref=recvbuf.at[h, me],
                    send_sem=send1.at[h, step],
                    recv_sem=recv1.at[h, me],
                    device_id=(tgt,),
                    device_id_type=pl.DeviceIdType.MESH,
                ).start()
                return 0
            lax.fori_loop(1, W, step_fn, 0)

        for h in range(H):
            p1_send(h)

        for h in range(H):
            accbuf[...] = sendbuf[
                pl.ds(me * rows, rows), pl.ds(h * nh, nh)
            ].astype(jnp.float32)

            def recv_acc(step, _):
                src = lax.rem(me - step + W, W)
                pltpu.make_async_remote_copy(
                    src_ref=sendbuf.at[pl.ds(0, rows), pl.ds(0, nh)],
                    dst_ref=recvbuf.at[h, src],
                    send_sem=send1.at[h, 0],
                    recv_sem=recv1.at[h, src],
                    device_id=(me,),
                    device_id_type=pl.DeviceIdType.MESH,
                ).wait_recv()
                accbuf[...] += recvbuf[h, src].astype(jnp.float32)
                return 0
            lax.fori_loop(1, W, recv_acc, 0)

            z = accbuf[...]
            y = z / (1.0 + jnp.exp(-z))
            out_ref[pl.ds(me * rows, rows), pl.ds(h * nh, nh)] = y.astype(
                jnp.bfloat16
            )

            def ag_send(step, _):
                tgt = lax.rem(me + step, W)
                pltpu.make_async_remote_copy(
                    src_ref=out_ref.at[pl.ds(me * rows, rows), pl.ds(h * nh, nh)],
                    dst_ref=out_ref.at[pl.ds(me * rows, rows), pl.ds(h * nh, nh)],
                    send_sem=send2.at[h, step],
                    recv_sem=recv2.at[h, me],
                    device_id=(tgt,),
                    device_id_type=pl.DeviceIdType.MESH,
                ).start()
                return 0
            lax.fori_loop(1, W, ag_send, 0)

        for h in range(H):
            def ag_recv(step, _):
                src = lax.rem(me - step + W, W)
                pltpu.make_async_remote_copy(
                    src_ref=out_ref.at[pl.ds(src * rows, rows), pl.ds(h * nh, nh)],
                    dst_ref=out_ref.at[pl.ds(src * rows, rows), pl.ds(h * nh, nh)],
                    send_sem=send2.at[h, 0],
                    recv_sem=recv2.at[h, src],
                    device_id=(me,),
                    device_id_type=pl.DeviceIdType.MESH,
                ).wait_recv()
                return 0
            lax.fori_loop(1, W, ag_recv, 0)

        for h in range(H):
            def drain_p1(step, _):
                pltpu.make_async_remote_copy(
                    src_ref=sendbuf.at[pl.ds(0, rows), pl.ds(0, nh)],
                    dst_ref=recvbuf.at[h, 0],
                    send_sem=send1.at[h, step],
                    recv_sem=recv1.at[h, 0],
                    device_id=(me,),
                    device_id_type=pl.DeviceIdType.MESH,
                ).wait_send()
                return 0
            lax.fori_loop(1, W, drain_p1, 0)

            def drain_p2(step, _):
                pltpu.make_async_remote_copy(
                    src_ref=out_ref.at[pl.ds(0, rows), pl.ds(h * nh, nh)],
                    dst_ref=out_ref.at[pl.ds(0, rows), pl.ds(h * nh, nh)],
                    send_sem=send2.at[h, step],
                    recv_sem=recv2.at[h, 0],
                    device_id=(me,),
                    device_id_type=pl.DeviceIdType.MESH,
                ).wait_send()
                return 0
            lax.fori_loop(1, W, drain_p2, 0)

    return pl.pallas_call(
        body,
        out_shape=jax.ShapeDtypeStruct((M, N), jnp.bfloat16),
        in_specs=[
            pl.BlockSpec(memory_space=pltpu.VMEM),
            pl.BlockSpec(memory_space=pltpu.VMEM),
        ],
        out_specs=pl.BlockSpec(memory_space=pltpu.VMEM),
        scratch_shapes=[
            pltpu.VMEM((M, N), jnp.bfloat16),
            pltpu.VMEM((H, W, rows, nh), jnp.bfloat16),
            pltpu.VMEM((rows, nh), jnp.float32),
            pltpu.SemaphoreType.DMA((H, W)),
            pltpu.SemaphoreType.DMA((H, W)),
            pltpu.SemaphoreType.DMA((H, W)),
            pltpu.SemaphoreType.DMA((H, W)),
        ],
    )(A, B)
